# baseline (speedup 1.0000x reference)
import functools

import jax
import jax.numpy as jnp
from jax import lax
from jax.experimental import pallas as pl
from jax.experimental.pallas import tpu as pltpu
from jax.experimental.pallas import tpu_sc as plsc

B = 4096
D = 128
H = 512


def _gather_sc(user_ids, item_ids, user_table, item_table):
    info = plsc.get_sparse_core_info()
    bpw = B // (info.num_cores * info.num_subcores)
    mesh = plsc.VectorSubcoreMesh(core_axis_name="c", subcore_axis_name="s")

    @functools.partial(
        pl.kernel,
        mesh=mesh,
        out_type=(jax.ShapeDtypeStruct((B, D), jnp.float32),
                  jax.ShapeDtypeStruct((B, D), jnp.float32)),
        scratch_types=[
            pltpu.VMEM((bpw,), jnp.int32),
            pltpu.VMEM((bpw,), jnp.int32),
            pltpu.VMEM((bpw, D), jnp.float32),
            pltpu.VMEM((bpw, D), jnp.float32),
            pltpu.SemaphoreType.DMA,
            pltpu.SemaphoreType.DMA,
            pltpu.SemaphoreType.DMA,
            pltpu.SemaphoreType.DMA,
        ],
    )
    def k(uids_hbm, iids_hbm, ut_hbm, it_hbm, uout_hbm, vout_hbm,
          uidx, iidx, urows, vrows, sem_a, sem_b, sem_c, sem_d):
        wid = lax.axis_index("s") * info.num_cores + lax.axis_index("c")
        base = wid * bpw
        cu_idx = pltpu.async_copy(uids_hbm.at[pl.ds(base, bpw)], uidx, sem_a)
        cv_idx = pltpu.async_copy(iids_hbm.at[pl.ds(base, bpw)], iidx, sem_b)
        cu_idx.wait()
        gu = pltpu.async_copy(ut_hbm.at[uidx], urows, sem_a)
        cv_idx.wait()
        gv = pltpu.async_copy(it_hbm.at[iidx], vrows, sem_b)
        gu.wait()
        wu = pltpu.async_copy(urows, uout_hbm.at[pl.ds(base, bpw)], sem_c)
        gv.wait()
        wv = pltpu.async_copy(vrows, vout_hbm.at[pl.ds(base, bpw)], sem_d)
        wu.wait()
        wv.wait()

    return k(user_ids, item_ids, user_table, item_table)


def _mlp_body(u_ref, v_ref, w1_ref, b1_ref, w2_ref, b2_ref, out_ref):
    x = (u_ref[...] * v_ref[...]).astype(jnp.bfloat16)
    h = jnp.dot(x, w1_ref[...], preferred_element_type=jnp.float32)
    h = jnp.maximum(h + b1_ref[...], 0.0).astype(jnp.bfloat16)
    out = jnp.dot(h, w2_ref[...], preferred_element_type=jnp.float32)
    out_ref[...] = out + b2_ref[...]


def _mlp_tc(u, v, W1, b1, W2, b2):
    blk = 4096
    return pl.pallas_call(
        _mlp_body,
        grid=(B // blk,),
        in_specs=[
            pl.BlockSpec((blk, D), lambda i: (i, 0)),
            pl.BlockSpec((blk, D), lambda i: (i, 0)),
            pl.BlockSpec((D, H), lambda i: (0, 0)),
            pl.BlockSpec((1, H), lambda i: (0, 0)),
            pl.BlockSpec((H, H), lambda i: (0, 0)),
            pl.BlockSpec((1, H), lambda i: (0, 0)),
        ],
        out_specs=pl.BlockSpec((blk, H), lambda i: (i, 0)),
        out_shape=jax.ShapeDtypeStruct((B, H), jnp.float32),
    )(u, v, W1.astype(jnp.bfloat16), b1, W2.astype(jnp.bfloat16), b2)


def kernel(user_ids, item_ids, user_table, item_table, W1, b1, W2, b2):
    u, v = _gather_sc(user_ids, item_ids, user_table, item_table)
    return _mlp_tc(u, v, W1, b1.reshape(1, H), W2, b2.reshape(1, H))


# speedup vs baseline: 1.0435x; 1.0435x over previous
import functools

import jax
import jax.numpy as jnp
from jax import lax
from jax.experimental import pallas as pl
from jax.experimental.pallas import tpu as pltpu
from jax.experimental.pallas import tpu_sc as plsc

B = 4096
D = 128
H = 512


def _gather_sc(user_ids, item_ids, user_table, item_table):
    info = plsc.get_sparse_core_info()
    bpw = B // (info.num_cores * info.num_subcores)
    mesh = plsc.VectorSubcoreMesh(core_axis_name="c", subcore_axis_name="s")

    @functools.partial(
        pl.kernel,
        mesh=mesh,
        out_type=(jax.ShapeDtypeStruct((B, D), jnp.float32),
                  jax.ShapeDtypeStruct((B, D), jnp.float32)),
        scratch_types=[
            pltpu.VMEM((bpw,), jnp.int32),
            pltpu.VMEM((bpw,), jnp.int32),
            pltpu.VMEM((bpw, D), jnp.float32),
            pltpu.VMEM((bpw, D), jnp.float32),
            pltpu.SemaphoreType.DMA,
            pltpu.SemaphoreType.DMA,
            pltpu.SemaphoreType.DMA,
            pltpu.SemaphoreType.DMA,
        ],
    )
    def k(uids_hbm, iids_hbm, ut_hbm, it_hbm, uout_hbm, vout_hbm,
          uidx, iidx, urows, vrows, sem_a, sem_b, sem_c, sem_d):
        wid = lax.axis_index("s") * info.num_cores + lax.axis_index("c")
        base = wid * bpw
        cu_idx = pltpu.async_copy(uids_hbm.at[pl.ds(base, bpw)], uidx, sem_a)
        cv_idx = pltpu.async_copy(iids_hbm.at[pl.ds(base, bpw)], iidx, sem_b)
        cu_idx.wait()
        gu = pltpu.async_copy(ut_hbm.at[uidx], urows, sem_a)
        cv_idx.wait()
        gv = pltpu.async_copy(it_hbm.at[iidx], vrows, sem_b)
        gu.wait()
        wu = pltpu.async_copy(urows, uout_hbm.at[pl.ds(base, bpw)], sem_c)
        gv.wait()
        wv = pltpu.async_copy(vrows, vout_hbm.at[pl.ds(base, bpw)], sem_d)
        wu.wait()
        wv.wait()

    return k(user_ids, item_ids, user_table, item_table)


def _mlp_body(u_ref, v_ref, w1_ref, b1_ref, w2_ref, b2_ref, out_ref):
    half = u_ref.shape[0] // 2
    for p in range(2):
        sl = pl.ds(p * half, half)
        x = (u_ref[sl, :] * v_ref[sl, :]).astype(jnp.bfloat16)
        h = jnp.dot(x, w1_ref[...], preferred_element_type=jnp.float32)
        h = jnp.maximum(h + b1_ref[...], 0.0).astype(jnp.bfloat16)
        out = jnp.dot(h, w2_ref[...], preferred_element_type=jnp.float32)
        out_ref[sl, :] = out + b2_ref[...]


def _mlp_tc(u, v, W1, b1, W2, b2):
    blk = 2048
    return pl.pallas_call(
        _mlp_body,
        grid=(B // blk,),
        in_specs=[
            pl.BlockSpec((blk, D), lambda i: (i, 0)),
            pl.BlockSpec((blk, D), lambda i: (i, 0)),
            pl.BlockSpec((D, H), lambda i: (0, 0)),
            pl.BlockSpec((1, H), lambda i: (0, 0)),
            pl.BlockSpec((H, H), lambda i: (0, 0)),
            pl.BlockSpec((1, H), lambda i: (0, 0)),
        ],
        out_specs=pl.BlockSpec((blk, H), lambda i: (i, 0)),
        out_shape=jax.ShapeDtypeStruct((B, H), jnp.float32),
    )(u, v, W1.astype(jnp.bfloat16), b1, W2.astype(jnp.bfloat16), b2)


def kernel(user_ids, item_ids, user_table, item_table, W1, b1, W2, b2):
    u, v = _gather_sc(user_ids, item_ids, user_table, item_table)
    return _mlp_tc(u, v, W1, b1.reshape(1, H), W2, b2.reshape(1, H))
